# Initial kernel scaffold; baseline (speedup 1.0000x reference)
import jax
import jax.numpy as jnp
from jax.experimental import pallas as pl


def kernel(x, edge_index, W1a, b1a, g1, be1, W1b, b1b, W2a, b2a, g2, be2, W2b, b2b, W3a, b3a, g3, be3, W3b, b3b, Wl1, bl1, Wl2, bl2):
    raise NotImplementedError("write your pallas kernel here")


# speedup vs baseline: 2.5466x; 2.5466x over previous
import functools

import jax
import jax.numpy as jnp
from jax import lax
from jax.experimental import pallas as pl
from jax.experimental.pallas import tpu as pltpu
from jax.experimental.pallas import tpu_sc as plsc

N_REAL = 10000
CH = 128
N_PAD = 10240
N_SUB = 16
EB = 128
WB = 8
BN = 1024
NBLK = N_PAD // BN
BNC = 512
NBLKC = N_PAD // BNC
ROWS_PER_SUB = N_PAD // N_SUB

_MESH_CACHE = []


def _mesh():
    if not _MESH_CACHE:
        _MESH_CACHE.append(plsc.VectorSubcoreMesh(
            core_axis_name="core", subcore_axis_name="subcore"))
    return _MESH_CACHE[0]


def _sc_agg_pair(x0, x1, src_b, dst_b, nb):
    out_type = (jax.ShapeDtypeStruct((N_PAD, CH), jnp.float32),
                jax.ShapeDtypeStruct((N_PAD, CH), jnp.float32))
    nw = nb // WB
    assert nb % (2 * WB) == 0 and nw % 2 == 0
    scratch = [
        pltpu.VMEM((2, WB, EB), jnp.int32),
        pltpu.VMEM((2, WB, EB), jnp.int32),
        pltpu.VMEM((EB, CH), jnp.float32),
        pltpu.VMEM((EB, CH), jnp.float32),
        pltpu.VMEM_SHARED((N_PAD, CH), jnp.float32),
        pltpu.SemaphoreType.DMA,
        pltpu.SemaphoreType.DMA,
        pltpu.SemaphoreType.DMA,
        pltpu.SemaphoreType.DMA,
    ]

    @functools.partial(pl.kernel, out_type=out_type, mesh=_mesh(),
                       scratch_types=scratch)
    def k(x0_hbm, x1_hbm, src_hbm, dst_hbm, o0_hbm, o1_hbm,
          src_w, dst_w, gbuf_a, gbuf_b, acc, sem_a, sem_b, sem_s, sem_d):
        c = lax.axis_index("core")
        s = lax.axis_index("subcore")

        @pl.loop(0, EB)
        def _(r):
            @pl.loop(0, CH, step=16)
            def _(q):
                gbuf_a[r, pl.ds(q, 16)] = jnp.zeros((16,), jnp.float32)

        @pl.loop(0, ROWS_PER_SUB, step=EB)
        def _(r0):
            pltpu.sync_copy(gbuf_a, acc.at[pl.ds(s * ROWS_PER_SUB + r0, EB)])

        plsc.subcore_barrier()

        def run(x_hbm):
            pltpu.sync_copy(src_hbm.at[s, pl.ds(0, WB)], src_w.at[0])
            pltpu.sync_copy(dst_hbm.at[s, pl.ds(0, WB)], dst_w.at[0])
            pltpu.async_copy(x_hbm.at[src_w.at[0, 0]], gbuf_a, sem_a)

            def do_window(w, wb, wbn):
                @pl.when(w + 1 < nw)
                def _():
                    pltpu.async_copy(src_hbm.at[s, pl.ds((w + 1) * WB, WB)],
                                     src_w.at[wbn], sem_s)
                    pltpu.async_copy(dst_hbm.at[s, pl.ds((w + 1) * WB, WB)],
                                     dst_w.at[wbn], sem_d)
                for p in range(WB // 2):
                    j0 = 2 * p
                    pltpu.make_async_copy(x_hbm.at[src_w.at[wb, j0]], gbuf_a,
                                          sem_a).wait()
                    pltpu.async_copy(x_hbm.at[src_w.at[wb, j0 + 1]], gbuf_b,
                                     sem_b)
                    pltpu.sync_copy(gbuf_a, acc.at[dst_w.at[wb, j0]], add=True)
                    pltpu.make_async_copy(x_hbm.at[src_w.at[wb, j0 + 1]],
                                          gbuf_b, sem_b).wait()
                    if j0 + 2 < WB:
                        pltpu.async_copy(x_hbm.at[src_w.at[wb, j0 + 2]],
                                         gbuf_a, sem_a)
                    else:
                        @pl.when(w + 1 < nw)
                        def _():
                            pltpu.make_async_copy(
                                src_hbm.at[s, pl.ds((w + 1) * WB, WB)],
                                src_w.at[wbn], sem_s).wait()
                            pltpu.make_async_copy(
                                dst_hbm.at[s, pl.ds((w + 1) * WB, WB)],
                                dst_w.at[wbn], sem_d).wait()
                            pltpu.async_copy(x_hbm.at[src_w.at[wbn, 0]],
                                             gbuf_a, sem_a)
                    pltpu.sync_copy(gbuf_b, acc.at[dst_w.at[wb, j0 + 1]],
                                    add=True)

            @pl.loop(0, nw, step=2)
            def _(w2):
                do_window(w2, 0, 1)
                do_window(w2 + 1, 1, 0)

        @pl.when(c == 0)
        def _():
            run(x0_hbm)

        @pl.when(c == 1)
        def _():
            run(x1_hbm)

        plsc.subcore_barrier()

        @pl.when(c == 0)
        def _():
            pltpu.sync_copy(acc.at[pl.ds(s * ROWS_PER_SUB, ROWS_PER_SUB)],
                            o0_hbm.at[pl.ds(s * ROWS_PER_SUB, ROWS_PER_SUB)])

        @pl.when(c == 1)
        def _():
            pltpu.sync_copy(acc.at[pl.ds(s * ROWS_PER_SUB, ROWS_PER_SUB)],
                            o1_hbm.at[pl.ds(s * ROWS_PER_SUB, ROWS_PER_SUB)])

    return k(x0, x1, src_b, dst_b)


def _mlp_stage1(x_chunks, agg_chunks, Wa, ba):
    C = len(x_chunks)
    D = C * CH
    H = Wa.shape[1]

    def body(*refs):
        xs = refs[:C]
        ags = refs[C:2 * C]
        wa_ref, ba_ref = refs[2 * C], refs[2 * C + 1]
        t_ref, ps_ref, pq_ref = refs[2 * C + 2:]
        i = pl.program_id(0)
        h = jnp.concatenate([xs[c][...] + ags[c][...] for c in range(C)],
                            axis=1)
        t = jnp.dot(h, wa_ref[...], preferred_element_type=jnp.float32)
        t = t + ba_ref[...]
        rows = i * BN + lax.broadcasted_iota(jnp.int32, (BN, 1), 0)
        tm = jnp.where(rows < N_REAL, t, 0.0)
        t_ref[...] = t
        ps_ref[...] = jnp.sum(tm, axis=0)[None, None, :]
        pq_ref[...] = jnp.sum(tm * tm, axis=0)[None, None, :]

    in_specs = ([pl.BlockSpec((BN, CH), lambda i: (i, 0))] * (2 * C)
                + [pl.BlockSpec((D, H), lambda i: (0, 0)),
                   pl.BlockSpec((1, H), lambda i: (0, 0))])
    out_specs = [pl.BlockSpec((BN, H), lambda i: (i, 0)),
                 pl.BlockSpec((1, 1, H), lambda i: (i, 0, 0)),
                 pl.BlockSpec((1, 1, H), lambda i: (i, 0, 0))]
    out_shape = [jax.ShapeDtypeStruct((N_PAD, H), jnp.float32),
                 jax.ShapeDtypeStruct((NBLK, 1, H), jnp.float32),
                 jax.ShapeDtypeStruct((NBLK, 1, H), jnp.float32)]
    return pl.pallas_call(body, grid=(NBLK,), in_specs=in_specs,
                          out_specs=out_specs, out_shape=out_shape)(
        *x_chunks, *agg_chunks, Wa, ba)


def _mlp_stage2(t, ps, pq, g, be, Wb, bb):
    H = t.shape[1]
    H2 = Wb.shape[1]
    C_out = H2 // CH

    def body(t_ref, ps_ref, pq_ref, g_ref, be_ref, wb_ref, bb_ref, *outs):
        mu = jnp.sum(ps_ref[...], axis=(0, 1)) / N_REAL
        ex2 = jnp.sum(pq_ref[...], axis=(0, 1)) / N_REAL
        inv = lax.rsqrt(jnp.maximum(ex2 - mu * mu, 0.0) + 1e-5)
        h = (t_ref[...] - mu) * (inv * g_ref[0]) + be_ref[0]
        h = jnp.maximum(h, 0.0)
        o = jnp.dot(h, wb_ref[...], preferred_element_type=jnp.float32)
        o = jnp.maximum(o + bb_ref[...], 0.0)
        for c in range(C_out):
            outs[c][...] = o[:, c * CH:(c + 1) * CH]

    in_specs = [pl.BlockSpec((BN, H), lambda i: (i, 0)),
                pl.BlockSpec((NBLK, 1, H), lambda i: (0, 0, 0)),
                pl.BlockSpec((NBLK, 1, H), lambda i: (0, 0, 0)),
                pl.BlockSpec((1, H), lambda i: (0, 0)),
                pl.BlockSpec((1, H), lambda i: (0, 0)),
                pl.BlockSpec((H, H2), lambda i: (0, 0)),
                pl.BlockSpec((1, H2), lambda i: (0, 0))]
    out_specs = [pl.BlockSpec((BN, CH), lambda i: (i, 0))] * C_out
    out_shape = [jax.ShapeDtypeStruct((N_PAD, CH), jnp.float32)] * C_out
    return pl.pallas_call(body, grid=(NBLK,), in_specs=in_specs,
                          out_specs=out_specs, out_shape=out_shape)(
        t, ps, pq, g, be, Wb, bb)


def _readout(h_chunks, Wl1, bl1, Wl2, bl2):
    C = len(h_chunks)
    D3 = C * CH
    DOUT = Wl2.shape[1]

    def body(*refs):
        hs = refs[:C]
        wl1_ref, bl1_ref, wl2_ref, bl2_ref, o_ref = refs[C:]
        hcat = jnp.concatenate([h[...] for h in hs], axis=1)
        u = jnp.dot(hcat, wl1_ref[...], preferred_element_type=jnp.float32)
        u = jnp.maximum(u + bl1_ref[...], 0.0)
        o = jnp.dot(u, wl2_ref[...], preferred_element_type=jnp.float32)
        o_ref[...] = o + bl2_ref[...]

    in_specs = ([pl.BlockSpec((BNC, CH), lambda i: (i, 0))] * C
                + [pl.BlockSpec((D3, D3), lambda i: (0, 0)),
                   pl.BlockSpec((1, D3), lambda i: (0, 0)),
                   pl.BlockSpec((D3, DOUT), lambda i: (0, 0)),
                   pl.BlockSpec((1, DOUT), lambda i: (0, 0))])
    out_specs = [pl.BlockSpec((BNC, DOUT), lambda i: (i, 0))]
    out_shape = [jax.ShapeDtypeStruct((N_PAD, DOUT), jnp.float32)]
    return pl.pallas_call(body, grid=(NBLKC,), in_specs=in_specs,
                          out_specs=out_specs, out_shape=out_shape)(
        *h_chunks, Wl1, bl1, Wl2, bl2)[0]


def kernel(x, edge_index, W1a, b1a, g1, be1, W1b, b1b, W2a, b2a, g2, be2,
           W2b, b2b, W3a, b3a, g3, be3, W3b, b3b, Wl1, bl1, Wl2, bl2):
    E = edge_index.shape[1]
    e_pad = -E % (N_SUB * EB * 2 * WB)
    e_tot = E + e_pad
    nb = e_tot // (N_SUB * EB)
    src = jnp.concatenate([edge_index[0], jnp.zeros((e_pad,), jnp.int32)])
    dst = jnp.concatenate([edge_index[1],
                           jnp.full((e_pad,), N_REAL, jnp.int32)])
    src_b = src.reshape(N_SUB, nb, EB)
    dst_b = dst.reshape(N_SUB, nb, EB)

    xp = jnp.pad(x, ((0, N_PAD - x.shape[0]), (0, 0)))
    x_chunks = [xp[:, c * CH:(c + 1) * CH] for c in range(x.shape[1] // CH)]

    def gin_layer(h_chunks, Wa, ba, g, be, Wb, bb):
        agg = []
        for c0 in range(0, len(h_chunks), 2):
            a0, a1 = _sc_agg_pair(h_chunks[c0], h_chunks[c0 + 1],
                                  src_b, dst_b, nb)
            agg += [a0, a1]
        t, ps, pq = _mlp_stage1(h_chunks, agg, Wa, ba.reshape(1, -1))
        return _mlp_stage2(t, ps, pq, g.reshape(1, -1), be.reshape(1, -1),
                           Wb, bb.reshape(1, -1))

    h1 = gin_layer(x_chunks, W1a, b1a, g1, be1, W1b, b1b)
    h2 = gin_layer(h1, W2a, b2a, g2, be2, W2b, b2b)
    h3 = gin_layer(h2, W3a, b3a, g3, be3, W3b, b3b)
    out = _readout(h1 + h2 + h3, Wl1, bl1.reshape(1, -1),
                   Wl2, bl2.reshape(1, -1))
    return out[:N_REAL]
